# baseline (speedup 1.0000x reference)
import functools

import jax
import jax.numpy as jnp
from jax import lax
from jax.experimental import pallas as pl
from jax.experimental.pallas import tpu as pltpu
from jax.experimental.pallas import tpu_sc as plsc

NUM_EMB = 1000000
EMB_DIM = 64
SUB_DIM = 32
ROW_PACK = 4
SUPER_ROWS = NUM_EMB // ROW_PACK
PAD_DIM = 128
BASE_SEED = 42
N = 4096 * 26
NW = 32
NPW = N // NW
CHUNK = 128
NCH = NPW // CHUNK
VPC = CHUNK // 16

_MAGIC_HI = 1125899907 >> 16
_MAGIC_LO = 1125899907 & 0xFFFF


def _umod_1e6(h):
    al = h & jnp.uint32(0xFFFF)
    ah = h >> 16
    t = ah * jnp.uint32(_MAGIC_LO) + ((al * jnp.uint32(_MAGIC_LO)) >> 16)
    t2 = al * jnp.uint32(_MAGIC_HI) + (t & jnp.uint32(0xFFFF))
    hi = ah * jnp.uint32(_MAGIC_HI) + (t >> 16) + (t2 >> 16)
    q = hi >> 18
    return h - q * jnp.uint32(NUM_EMB)


def _hash16(v_u32, seed):
    h = v_u32 ^ jnp.uint32(seed)
    h = h * jnp.uint32(2654435761)
    h = h ^ (h >> 16)
    h = h * jnp.uint32(2246822519)
    h = h ^ (h >> 13)
    return _umod_1e6(h)


def _emb_body(ids_hbm, w0_hbm, w1_hbm, out_hbm,
              ids_v, idx0_v, idx1_v, off0_v, off1_v,
              g0a, g0b, g1a, g1b, obuf,
              s0a, s0b, s1a, s1b):
    wid = lax.axis_index("s") * 2 + lax.axis_index("c")
    base = wid * NPW

    pltpu.sync_copy(ids_hbm.at[wid], ids_v)

    def hash_chunk(c):
        for j in range(VPC):
            u = plsc.bitcast(ids_v[c, pl.ds(j * 16, 16)], jnp.uint32)
            h0 = _hash16(u, BASE_SEED)
            h1 = _hash16(u, BASE_SEED + 1)
            sl = pl.ds(j * 16, 16)
            idx0_v[c, sl] = plsc.bitcast(h0 >> 2, jnp.int32)
            idx1_v[c, sl] = plsc.bitcast(h1 >> 2, jnp.int32)
            off0_v[c, sl] = plsc.bitcast((h0 & jnp.uint32(3)) << 5, jnp.int32)
            off1_v[c, sl] = plsc.bitcast((h1 & jnp.uint32(3)) << 5, jnp.int32)

    lax.fori_loop(0, NCH, lambda c, _: (hash_chunk(c), 0)[1], 0, unroll=2)

    gbufs = ((g0a, g1a, s0a, s1a), (g0b, g1b, s0b, s1b))

    def issue(c, p):
        g0, g1, s0, s1 = gbufs[p]
        pltpu.async_copy(w0_hbm.at[idx0_v.at[c]], g0, s0)
        pltpu.async_copy(w1_hbm.at[idx1_v.at[c]], g1, s1)

    def wait(p):
        g0, g1, s0, s1 = gbufs[p]
        pltpu.make_async_copy(w0_hbm.at[idx0_v.at[0]], g0, s0).wait()
        pltpu.make_async_copy(w1_hbm.at[idx1_v.at[0]], g1, s1).wait()

    def extract(c, p):
        g0, g1, _, _ = gbufs[p]

        def group(t, _):
            rb = t * 16
            ov0 = off0_v[c, pl.ds(rb, 16)]
            ov1 = off1_v[c, pl.ds(rb, 16)]
            for l in range(16):
                r = rb + l
                o0 = ov0[l]
                o1 = ov1[l]
                obuf[r, pl.ds(0, 16)] = g0[r, pl.ds(o0, 16)]
                obuf[r, pl.ds(16, 16)] = g0[r, pl.ds(o0 + 16, 16)]
                obuf[r, pl.ds(32, 16)] = g1[r, pl.ds(o1, 16)]
                obuf[r, pl.ds(48, 16)] = g1[r, pl.ds(o1 + 16, 16)]
            return 0

        lax.fori_loop(0, CHUNK // 16, group, 0)

    issue(0, 0)

    def pair_body(i, _):
        for k in range(2):
            c = i * 2 + k
            p = k
            wait(p)

            @pl.when(c + 1 < NCH)
            def _():
                issue(c + 1, 1 - p)

            extract(c, p)
            pltpu.sync_copy(obuf, out_hbm.at[pl.ds(base + c * CHUNK, CHUNK)])
        return 0

    lax.fori_loop(0, NCH // 2, pair_body, 0)


_emb = functools.partial(
    pl.kernel,
    out_type=jax.ShapeDtypeStruct((N, EMB_DIM), jnp.float32),
    mesh=plsc.VectorSubcoreMesh(core_axis_name="c", subcore_axis_name="s"),
    compiler_params=pltpu.CompilerParams(use_tc_tiling_on_sc=True),
    scratch_types=[
        pltpu.VMEM((NCH, CHUNK), jnp.int32),
        pltpu.VMEM((NCH, CHUNK), jnp.int32),
        pltpu.VMEM((NCH, CHUNK), jnp.int32),
        pltpu.VMEM((NCH, CHUNK), jnp.int32),
        pltpu.VMEM((NCH, CHUNK), jnp.int32),
        pltpu.VMEM((CHUNK, PAD_DIM), jnp.float32),
        pltpu.VMEM((CHUNK, PAD_DIM), jnp.float32),
        pltpu.VMEM((CHUNK, PAD_DIM), jnp.float32),
        pltpu.VMEM((CHUNK, PAD_DIM), jnp.float32),
        pltpu.VMEM((CHUNK, EMB_DIM), jnp.float32),
        pltpu.SemaphoreType.DMA,
        pltpu.SemaphoreType.DMA,
        pltpu.SemaphoreType.DMA,
        pltpu.SemaphoreType.DMA,
    ],
)(_emb_body)


@jax.jit
def kernel(input_ids, W0, W1):
    ids3d = input_ids.reshape(NW, NCH, CHUNK)
    w0f, w1f = jax.lax.optimization_barrier((W0.reshape(-1), W1.reshape(-1)))
    w0r = w0f.reshape(SUPER_ROWS, PAD_DIM)
    w1r = w1f.reshape(SUPER_ROWS, PAD_DIM)
    out = _emb(ids3d, w0r, w1r)
    return out.reshape(input_ids.shape + (EMB_DIM,))


# speedup vs baseline: 1.0489x; 1.0489x over previous
import functools

import jax
import jax.numpy as jnp
from jax import lax
from jax.experimental import pallas as pl
from jax.experimental.pallas import tpu as pltpu
from jax.experimental.pallas import tpu_sc as plsc

NUM_EMB = 1000000
EMB_DIM = 64
SUB_DIM = 32
BASE_SEED = 42
N = 4096 * 26
NW = 32
NPW = N // NW
CHUNK = 128
NCH = NPW // CHUNK
VPC = CHUNK // 16

_MAGIC_HI = 1125899907 >> 16
_MAGIC_LO = 1125899907 & 0xFFFF


def _umod_1e6(h):
    al = h & jnp.uint32(0xFFFF)
    ah = h >> 16
    t = ah * jnp.uint32(_MAGIC_LO) + ((al * jnp.uint32(_MAGIC_LO)) >> 16)
    t2 = al * jnp.uint32(_MAGIC_HI) + (t & jnp.uint32(0xFFFF))
    hi = ah * jnp.uint32(_MAGIC_HI) + (t >> 16) + (t2 >> 16)
    q = hi >> 18
    return h - q * jnp.uint32(NUM_EMB)


def _hash16(v_u32, seed):
    h = v_u32 ^ jnp.uint32(seed)
    h = h * jnp.uint32(2654435761)
    h = h ^ (h >> 16)
    h = h * jnp.uint32(2246822519)
    h = h ^ (h >> 13)
    return plsc.bitcast(_umod_1e6(h), jnp.int32)


def _emb_body(ids_hbm, w0_hbm, w1_hbm, out_hbm,
              ids_v, idx0_v, idx1_v, buf0, buf1, sem0, sem1):
    wid = lax.axis_index("s") * 2 + lax.axis_index("c")
    base = wid * NPW

    pltpu.sync_copy(ids_hbm.at[pl.ds(base, NPW)], ids_v)

    def hash_chunk(c):
        for j in range(VPC):
            v = ids_v[pl.ds(c * CHUNK + j * 16, 16)]
            u = plsc.bitcast(v, jnp.uint32)
            idx0_v[c, pl.ds(j * 16, 16)] = _hash16(u, BASE_SEED)
            idx1_v[c, pl.ds(j * 16, 16)] = _hash16(u, BASE_SEED + 1)

    lax.fori_loop(0, NCH, lambda c, _: (hash_chunk(c), 0)[1], 0)

    def gather_chunk(c, _):
        cp0 = pltpu.async_copy(w0_hbm.at[idx0_v.at[c]], buf0, sem0)
        cp1 = pltpu.async_copy(w1_hbm.at[idx1_v.at[c]], buf1, sem1)
        cp0.wait()
        cp1.wait()
        row = base + c * CHUNK
        pltpu.sync_copy(buf0, out_hbm.at[pl.ds(row, CHUNK), pl.ds(0, SUB_DIM)])
        pltpu.sync_copy(buf1, out_hbm.at[pl.ds(row, CHUNK), pl.ds(SUB_DIM, SUB_DIM)])
        return 0

    lax.fori_loop(0, NCH, gather_chunk, 0)


_emb = functools.partial(
    pl.kernel,
    out_type=jax.ShapeDtypeStruct((N, EMB_DIM), jnp.float32),
    mesh=plsc.VectorSubcoreMesh(core_axis_name="c", subcore_axis_name="s"),
    compiler_params=pltpu.CompilerParams(use_tc_tiling_on_sc=False),
    scratch_types=[
        pltpu.VMEM((NPW,), jnp.int32),
        pltpu.VMEM((NCH, CHUNK), jnp.int32),
        pltpu.VMEM((NCH, CHUNK), jnp.int32),
        pltpu.VMEM((CHUNK, SUB_DIM), jnp.float32),
        pltpu.VMEM((CHUNK, SUB_DIM), jnp.float32),
        pltpu.SemaphoreType.DMA,
        pltpu.SemaphoreType.DMA,
    ],
)(_emb_body)


@jax.jit
def kernel(input_ids, W0, W1):
    flat = input_ids.reshape(-1)
    out = _emb(flat, W0, W1)
    return out.reshape(input_ids.shape + (EMB_DIM,))
